# baseline (speedup 1.0000x reference)
import functools

import jax
import jax.numpy as jnp
from jax import lax
from jax.experimental import pallas as pl
from jax.experimental.pallas import tpu as pltpu
from jax.experimental.pallas import tpu_sc as plsc

N = 16384
DIM = 2048
E = 64
K = 8
BN = 2048
L = 16
NW = 32
TPW = N // NW


def _mm_body(x_ref, w_ref, scores_ref, scores_t_ref):
    logits_t = lax.dot_general(
        w_ref[...], x_ref[...], (((1,), (1,)), ((), ())),
        preferred_element_type=jnp.float32)
    st = jax.nn.sigmoid(logits_t)
    scores_t_ref[...] = st
    scores_ref[...] = st.T


def _sc_route(scores_t_hbm, bias_hbm, tops_t_hbm, idx_t_hbm,
              sc_v, bias_v, tops_v, idx_v):
    c = lax.axis_index("c")
    s = lax.axis_index("s")
    wid = s * 2 + c
    base = wid * TPW
    pltpu.sync_copy(scores_t_hbm.at[:, pl.ds(base, TPW)], sc_v)
    pltpu.sync_copy(bias_hbm, bias_v)

    def group_body(g, carry):
        t0 = g * L
        keys = [jnp.full((L,), -jnp.inf, jnp.float32) for _ in range(K)]
        idxs = [jnp.zeros((L,), jnp.int32) for _ in range(K)]
        raws = [jnp.zeros((L,), jnp.float32) for _ in range(K)]
        for e in range(E):
            tr = sc_v[e, pl.ds(t0, L)]
            tk = tr + bias_v[e, :]
            ti = jnp.full((L,), e, jnp.int32)
            for j in range(K):
                cs = tk > keys[j]
                keys[j], tk = (jnp.where(cs, tk, keys[j]),
                               jnp.where(cs, keys[j], tk))
                idxs[j], ti = (jnp.where(cs, ti, idxs[j]),
                               jnp.where(cs, idxs[j], ti))
                raws[j], tr = (jnp.where(cs, tr, raws[j]),
                               jnp.where(cs, raws[j], tr))
        denom = raws[0]
        for j in range(1, K):
            denom = denom + raws[j]
        denom = denom + 1e-20
        for j in range(K):
            tops_v[j, pl.ds(t0, L)] = raws[j] / denom
            idx_v[j, pl.ds(t0, L)] = idxs[j]
        return carry

    lax.fori_loop(0, TPW // L, group_body, 0)

    pltpu.sync_copy(tops_v, tops_t_hbm.at[:, pl.ds(base, TPW)])
    pltpu.sync_copy(idx_v, idx_t_hbm.at[:, pl.ds(base, TPW)])


def _fin_body(topst_ref, idxt_ref, tops_ref, idx_ref, counts_ref, ent_ref,
              acc_ref):
    i = pl.program_id(0)
    nsteps = pl.num_programs(0)
    tt = topst_ref[...]
    it = idxt_ref[...]
    tops_ref[...] = tt.T
    idx_ref[...] = it.T

    iota64 = lax.broadcasted_iota(jnp.int32, (E, BN), 0)
    part = jnp.zeros((E, BN), jnp.float32)
    for j in range(K):
        part = part + (it[j:j + 1, :] == iota64).astype(jnp.float32)

    ent_part = jnp.sum(tt * jnp.log(tt))

    @pl.when(i == 0)
    def _init():
        acc_ref[...] = part
        ent_ref[...] = jnp.full((1, 1), ent_part, jnp.float32)

    @pl.when(i > 0)
    def _acc():
        acc_ref[...] += part
        ent_ref[...] += ent_part

    @pl.when(i == nsteps - 1)
    def _fin():
        counts_ref[...] = jnp.sum(acc_ref[...], axis=1, keepdims=True)
        ent_ref[...] = -ent_ref[...] * (1.0 / N)


@jax.jit
def kernel(x, expert_bias, W):
    grid = (N // BN,)
    scores, scores_t = pl.pallas_call(
        _mm_body,
        grid=grid,
        in_specs=[
            pl.BlockSpec((BN, DIM), lambda i: (i, 0)),
            pl.BlockSpec((E, DIM), lambda i: (0, 0)),
        ],
        out_specs=[
            pl.BlockSpec((BN, E), lambda i: (i, 0)),
            pl.BlockSpec((E, BN), lambda i: (0, i)),
        ],
        out_shape=[
            jax.ShapeDtypeStruct((N, E), jnp.float32),
            jax.ShapeDtypeStruct((E, N), jnp.float32),
        ],
    )(x, W)

    bias16 = jnp.broadcast_to(expert_bias.reshape(E, 1), (E, L))
    mesh = plsc.VectorSubcoreMesh(core_axis_name="c", subcore_axis_name="s")
    tops_t, idx_t = pl.kernel(
        _sc_route,
        mesh=mesh,
        out_type=[
            jax.ShapeDtypeStruct((K, N), jnp.float32),
            jax.ShapeDtypeStruct((K, N), jnp.int32),
        ],
        scratch_types=[
            pltpu.VMEM((E, TPW), jnp.float32),
            pltpu.VMEM((E, L), jnp.float32),
            pltpu.VMEM((K, TPW), jnp.float32),
            pltpu.VMEM((K, TPW), jnp.int32),
        ],
    )(scores_t, bias16)

    tops, idx, counts, ent = pl.pallas_call(
        _fin_body,
        grid=grid,
        in_specs=[
            pl.BlockSpec((K, BN), lambda i: (0, i)),
            pl.BlockSpec((K, BN), lambda i: (0, i)),
        ],
        out_specs=[
            pl.BlockSpec((BN, K), lambda i: (i, 0)),
            pl.BlockSpec((BN, K), lambda i: (i, 0)),
            pl.BlockSpec((E, 1), lambda i: (0, 0)),
            pl.BlockSpec((1, 1), lambda i: (0, 0)),
        ],
        out_shape=[
            jax.ShapeDtypeStruct((N, K), jnp.float32),
            jax.ShapeDtypeStruct((N, K), jnp.int32),
            jax.ShapeDtypeStruct((E, 1), jnp.float32),
            jax.ShapeDtypeStruct((1, 1), jnp.float32),
        ],
        scratch_shapes=[pltpu.VMEM((E, BN), jnp.float32)],
    )(tops_t, idx_t)

    return (tops, scores, idx, counts.reshape(E), ent.reshape(()))
